# initial kernel scaffold (unmeasured)
import functools

import jax
import jax.numpy as jnp
from jax import lax
from jax.experimental import pallas as pl
from jax.experimental.pallas import tpu as pltpu

N_DEV = 4


def _rs_body(partial_ref, out_ref, acc, stage, recv_hbm,
             send_sems, recv_sems, local_sem):
    my = lax.axis_index("i")
    left = (my - 1) % N_DEV
    right = (my + 1) % N_DEV
    n_chunks, m_per, n_out = partial_ref.shape
    m_half = m_per // 2

    barrier_sem = pltpu.get_barrier_semaphore()
    for nbr in (left, right):
        pl.semaphore_signal(barrier_sem, inc=1, device_id=(nbr,),
                            device_id_type=pl.DeviceIdType.MESH)
    pl.semaphore_wait(barrier_sem, 2)

    def load_chunk_add(c, first_tile_to_acc):
        for t in range(2):
            cp = pltpu.make_async_copy(
                partial_ref.at[c, pl.ds(t * m_half, m_half), :],
                stage, local_sem)
            cp.start()
            cp.wait()
            if first_tile_to_acc:
                acc[pl.ds(t * m_half, m_half), :] = stage[...]
            else:
                acc[pl.ds(t * m_half, m_half), :] = (
                    acc[pl.ds(t * m_half, m_half), :] + stage[...])

    load_chunk_add((my - 1) % N_DEV, True)

    for s in range(N_DEV - 1):
        rdma = pltpu.make_async_remote_copy(
            src_ref=acc,
            dst_ref=recv_hbm.at[s],
            send_sem=send_sems.at[s],
            recv_sem=recv_sems.at[s],
            device_id=(right,),
            device_id_type=pl.DeviceIdType.MESH,
        )
        rdma.start()
        rdma.wait()

        c = (my - 2 - s) % N_DEV
        cp = pltpu.make_async_copy(recv_hbm.at[s], acc, local_sem)
        cp.start()
        cp.wait()
        load_chunk_add(c, False)

    acc[...] = jnp.maximum(acc[...], 0.0)
    cp = pltpu.make_async_copy(acc, out_ref, local_sem)
    cp.start()
    cp.wait()

    @functools.partial(pl.run_scoped, second=pltpu.SemaphoreType.REGULAR)
    def _(second):
        for nbr in (left, right):
            pl.semaphore_signal(second, inc=1, device_id=(nbr,),
                                device_id_type=pl.DeviceIdType.MESH)
        pl.semaphore_wait(second, 2)


def kernel(x, w_mat):
    m, k_loc = x.shape
    _, n_out = w_mat.shape
    m_per = m // N_DEV

    partial = jnp.dot(x, w_mat, preferred_element_type=jnp.float32)
    partial = partial.reshape(N_DEV, m_per, n_out)

    return pl.pallas_call(
        _rs_body,
        out_shape=jax.ShapeDtypeStruct((m_per, n_out), jnp.float32),
        in_specs=[pl.BlockSpec(memory_space=pl.ANY)],
        out_specs=pl.BlockSpec(memory_space=pl.ANY),
        scratch_shapes=[
            pltpu.VMEM((m_per, n_out), jnp.float32),
            pltpu.VMEM((m_per // 2, n_out), jnp.float32),
            pltpu.HBM((N_DEV - 1, m_per, n_out), jnp.float32),
            pltpu.SemaphoreType.DMA((N_DEV - 1,)),
            pltpu.SemaphoreType.DMA((N_DEV - 1,)),
            pltpu.SemaphoreType.DMA,
        ],
        compiler_params=pltpu.CompilerParams(
            collective_id=0,
            vmem_limit_bytes=60 * 1024 * 1024,
        ),
    )(partial)


# baseline (device time: 1406750 ns/iter reference)
import functools

import jax
import jax.numpy as jnp
from jax import lax
from jax.experimental import pallas as pl
from jax.experimental.pallas import tpu as pltpu

N_DEV = 4


def _rs_body(partial_ref, out_ref, recv_hbm, acc, stage,
             send_sems, recv_sems, local_sem):
    my = lax.axis_index("i")
    left = (my - 1) % N_DEV
    right = (my + 1) % N_DEV
    n_chunks, m_per, n_out = partial_ref.shape
    m_half = m_per // 2

    barrier_sem = pltpu.get_barrier_semaphore()
    for nbr in (left, right):
        pl.semaphore_signal(barrier_sem, inc=1, device_id=(nbr,),
                            device_id_type=pl.DeviceIdType.MESH)
    pl.semaphore_wait(barrier_sem, 2)

    def load_chunk_add(c, first_tile_to_acc):
        for t in range(2):
            cp = pltpu.make_async_copy(
                partial_ref.at[c, pl.ds(t * m_half, m_half), :],
                stage, local_sem)
            cp.start()
            cp.wait()
            if first_tile_to_acc:
                acc[pl.ds(t * m_half, m_half), :] = stage[...]
            else:
                acc[pl.ds(t * m_half, m_half), :] = (
                    acc[pl.ds(t * m_half, m_half), :] + stage[...])

    load_chunk_add((my - 1) % N_DEV, True)

    for s in range(N_DEV - 1):
        rdma = pltpu.make_async_remote_copy(
            src_ref=acc,
            dst_ref=recv_hbm.at[s],
            send_sem=send_sems.at[s],
            recv_sem=recv_sems.at[s],
            device_id=(right,),
            device_id_type=pl.DeviceIdType.MESH,
        )
        rdma.start()
        rdma.wait()

        c = (my - 2 - s) % N_DEV
        cp = pltpu.make_async_copy(recv_hbm.at[s], acc, local_sem)
        cp.start()
        cp.wait()
        load_chunk_add(c, False)

    acc[...] = jnp.maximum(acc[...], 0.0)
    cp = pltpu.make_async_copy(acc, out_ref, local_sem)
    cp.start()
    cp.wait()

    @functools.partial(pl.run_scoped, second=pltpu.SemaphoreType.REGULAR)
    def _(second):
        for nbr in (left, right):
            pl.semaphore_signal(second, inc=1, device_id=(nbr,),
                                device_id_type=pl.DeviceIdType.MESH)
        pl.semaphore_wait(second, 2)


def kernel(x, w_mat):
    m, k_loc = x.shape
    _, n_out = w_mat.shape
    m_per = m // N_DEV

    partial = jnp.dot(x, w_mat, preferred_element_type=jnp.float32)
    partial = partial.reshape(N_DEV, m_per, n_out)

    out, _ = pl.pallas_call(
        _rs_body,
        out_shape=[
            jax.ShapeDtypeStruct((m_per, n_out), jnp.float32),
            jax.ShapeDtypeStruct((N_DEV - 1, m_per, n_out), jnp.float32),
        ],
        in_specs=[pl.BlockSpec(memory_space=pl.ANY)],
        out_specs=[
            pl.BlockSpec(memory_space=pl.ANY),
            pl.BlockSpec(memory_space=pl.ANY),
        ],
        scratch_shapes=[
            pltpu.VMEM((m_per, n_out), jnp.float32),
            pltpu.VMEM((m_per // 2, n_out), jnp.float32),
            pltpu.SemaphoreType.DMA((N_DEV - 1,)),
            pltpu.SemaphoreType.DMA((N_DEV - 1,)),
            pltpu.SemaphoreType.DMA,
        ],
        compiler_params=pltpu.CompilerParams(
            collective_id=0,
            vmem_limit_bytes=60 * 1024 * 1024,
        ),
    )(partial)
    return out


# device time: 874402 ns/iter; 1.6088x vs baseline; 1.6088x over previous
import functools

import jax
import jax.numpy as jnp
from jax import lax
from jax.experimental import pallas as pl
from jax.experimental.pallas import tpu as pltpu

N_DEV = 4
N_STEP = N_DEV - 1


def _rs_body(partial_ref, out_ref, recv_hbm, acc_r, acc_l, stage,
             send_sems, recv_sems, local_sem):
    my = lax.axis_index("i")
    left = (my - 1) % N_DEV
    right = (my + 1) % N_DEV
    _, m_per, n_out = partial_ref.shape
    n_half = n_out // 2
    m_half = m_per // 2

    barrier_sem = pltpu.get_barrier_semaphore()
    for nbr in (left, right):
        pl.semaphore_signal(barrier_sem, inc=1, device_id=(nbr,),
                            device_id_type=pl.DeviceIdType.MESH)
    pl.semaphore_wait(barrier_sem, 2)

    def copy(src, dst):
        cp = pltpu.make_async_copy(src, dst, local_sem)
        cp.start()
        cp.wait()

    def add_partial(acc, c, col_off):
        for t in range(2):
            rows = pl.ds(t * m_half, m_half)
            copy(partial_ref.at[c, rows, pl.ds(col_off, n_half)], stage)
            acc[rows, :] = acc[rows, :] + stage[...]

    copy(partial_ref.at[(my - 1) % N_DEV, :, pl.ds(0, n_half)], acc_r)
    copy(partial_ref.at[(my + 1) % N_DEV, :, pl.ds(n_half, n_half)], acc_l)

    for s in range(N_STEP):
        rdma_r = pltpu.make_async_remote_copy(
            src_ref=acc_r,
            dst_ref=recv_hbm.at[0, s],
            send_sem=send_sems.at[0, s],
            recv_sem=recv_sems.at[0, s],
            device_id=(right,),
            device_id_type=pl.DeviceIdType.MESH,
        )
        rdma_l = pltpu.make_async_remote_copy(
            src_ref=acc_l,
            dst_ref=recv_hbm.at[1, s],
            send_sem=send_sems.at[1, s],
            recv_sem=recv_sems.at[1, s],
            device_id=(left,),
            device_id_type=pl.DeviceIdType.MESH,
        )
        rdma_r.start()
        rdma_l.start()

        rdma_r.wait()
        c_r = (my - 2 - s) % N_DEV
        copy(recv_hbm.at[0, s], acc_r)
        add_partial(acc_r, c_r, 0)

        rdma_l.wait()
        c_l = (my + 2 + s) % N_DEV
        copy(recv_hbm.at[1, s], acc_l)
        add_partial(acc_l, c_l, n_half)

    acc_r[...] = jnp.maximum(acc_r[...], 0.0)
    acc_l[...] = jnp.maximum(acc_l[...], 0.0)
    copy(acc_r, out_ref.at[:, pl.ds(0, n_half)])
    copy(acc_l, out_ref.at[:, pl.ds(n_half, n_half)])

    @functools.partial(pl.run_scoped, second=pltpu.SemaphoreType.REGULAR)
    def _(second):
        for nbr in (left, right):
            pl.semaphore_signal(second, inc=1, device_id=(nbr,),
                                device_id_type=pl.DeviceIdType.MESH)
        pl.semaphore_wait(second, 2)


def kernel(x, w_mat):
    m, k_loc = x.shape
    _, n_out = w_mat.shape
    m_per = m // N_DEV
    n_half = n_out // 2

    partial = jnp.dot(x, w_mat, preferred_element_type=jnp.float32)
    partial = partial.reshape(N_DEV, m_per, n_out)

    out, _ = pl.pallas_call(
        _rs_body,
        out_shape=[
            jax.ShapeDtypeStruct((m_per, n_out), jnp.float32),
            jax.ShapeDtypeStruct((2, N_STEP, m_per, n_half), jnp.float32),
        ],
        in_specs=[pl.BlockSpec(memory_space=pl.ANY)],
        out_specs=[
            pl.BlockSpec(memory_space=pl.ANY),
            pl.BlockSpec(memory_space=pl.ANY),
        ],
        scratch_shapes=[
            pltpu.VMEM((m_per, n_half), jnp.float32),
            pltpu.VMEM((m_per, n_half), jnp.float32),
            pltpu.VMEM((m_per // 2, n_half), jnp.float32),
            pltpu.SemaphoreType.DMA((2, N_STEP)),
            pltpu.SemaphoreType.DMA((2, N_STEP)),
            pltpu.SemaphoreType.DMA,
        ],
        compiler_params=pltpu.CompilerParams(
            collective_id=0,
            vmem_limit_bytes=60 * 1024 * 1024,
        ),
    )(partial)
    return out


# device time: 721457 ns/iter; 1.9499x vs baseline; 1.2120x over previous
import functools

import jax
import jax.numpy as jnp
from jax import lax
from jax.experimental import pallas as pl
from jax.experimental.pallas import tpu as pltpu

N_DEV = 4
N_STEP = N_DEV - 1
K_TILES = 2


def _body(x_ref, w_ref, out_ref, recv_hbm, send_hbm,
          comp_r, comp_l, stage, xt, wt,
          send_sems, recv_sems, local_sem):
    my = lax.axis_index("i")
    left = (my - 1) % N_DEV
    right = (my + 1) % N_DEV
    m_per, n_half = comp_r.shape
    k_loc = x_ref.shape[1]
    k_tile = k_loc // K_TILES
    m_half = m_per // 2

    barrier_sem = pltpu.get_barrier_semaphore()
    for nbr in (left, right):
        pl.semaphore_signal(barrier_sem, inc=1, device_id=(nbr,),
                            device_id_type=pl.DeviceIdType.MESH)
    pl.semaphore_wait(barrier_sem, 2)

    def copy(src, dst):
        cp = pltpu.make_async_copy(src, dst, local_sem)
        cp.start()
        cp.wait()

    def compute_partial(comp, c, col_off):
        for kt in range(K_TILES):
            copy(x_ref.at[pl.ds(c * m_per, m_per), pl.ds(kt * k_tile, k_tile)],
                 xt)
            copy(w_ref.at[pl.ds(kt * k_tile, k_tile), pl.ds(col_off, n_half)],
                 wt)
            prod = jnp.dot(xt[...], wt[...],
                           preferred_element_type=jnp.float32)
            if kt == 0:
                comp[...] = prod
            else:
                comp[...] = comp[...] + prod

    def add_recv(comp, d, t):
        for r in range(2):
            rows = pl.ds(r * m_half, m_half)
            copy(recv_hbm.at[d, t, rows, :], stage)
            comp[rows, :] = comp[rows, :] + stage[...]

    def rdma(d, t, nbr):
        return pltpu.make_async_remote_copy(
            src_ref=send_hbm.at[d],
            dst_ref=recv_hbm.at[d, t],
            send_sem=send_sems.at[d, t],
            recv_sem=recv_sems.at[d, t],
            device_id=(nbr,),
            device_id_type=pl.DeviceIdType.MESH,
        )

    n_off_l = n_half

    def step(s, carry):
        t = s - 1
        compute_partial(comp_r, (my - 2 - t) % N_DEV, 0)

        @pl.when(t >= 0)
        def _():
            rdma(0, t, right).wait()
            add_recv(comp_r, 0, t)

        @pl.when(t < N_STEP - 1)
        def _():
            copy(comp_r, send_hbm.at[0])
            rdma(0, t + 1, right).start()

        compute_partial(comp_l, (my + 2 + t) % N_DEV, n_off_l)

        @pl.when(t >= 0)
        def _():
            rdma(1, t, left).wait()
            add_recv(comp_l, 1, t)

        @pl.when(t < N_STEP - 1)
        def _():
            copy(comp_l, send_hbm.at[1])
            rdma(1, t + 1, left).start()

        return carry

    lax.fori_loop(0, N_DEV, step, 0)

    comp_r[...] = jnp.maximum(comp_r[...], 0.0)
    comp_l[...] = jnp.maximum(comp_l[...], 0.0)
    copy(comp_r, out_ref.at[:, pl.ds(0, n_half)])
    copy(comp_l, out_ref.at[:, pl.ds(n_off_l, n_half)])

    @functools.partial(pl.run_scoped, second=pltpu.SemaphoreType.REGULAR)
    def _(second):
        for nbr in (left, right):
            pl.semaphore_signal(second, inc=1, device_id=(nbr,),
                                device_id_type=pl.DeviceIdType.MESH)
        pl.semaphore_wait(second, 2)


def kernel(x, w_mat):
    m, k_loc = x.shape
    _, n_out = w_mat.shape
    m_per = m // N_DEV
    n_half = n_out // 2
    k_tile = k_loc // K_TILES

    out, _, _ = pl.pallas_call(
        _body,
        out_shape=[
            jax.ShapeDtypeStruct((m_per, n_out), jnp.float32),
            jax.ShapeDtypeStruct((2, N_STEP, m_per, n_half), jnp.float32),
            jax.ShapeDtypeStruct((2, m_per, n_half), jnp.float32),
        ],
        in_specs=[
            pl.BlockSpec(memory_space=pl.ANY),
            pl.BlockSpec(memory_space=pl.ANY),
        ],
        out_specs=[
            pl.BlockSpec(memory_space=pl.ANY),
            pl.BlockSpec(memory_space=pl.ANY),
            pl.BlockSpec(memory_space=pl.ANY),
        ],
        scratch_shapes=[
            pltpu.VMEM((m_per, n_half), jnp.float32),
            pltpu.VMEM((m_per, n_half), jnp.float32),
            pltpu.VMEM((m_per // 2, n_half), jnp.float32),
            pltpu.VMEM((m_per, k_tile), jnp.float32),
            pltpu.VMEM((k_tile, n_half), jnp.float32),
            pltpu.SemaphoreType.DMA((2, N_STEP)),
            pltpu.SemaphoreType.DMA((2, N_STEP)),
            pltpu.SemaphoreType.DMA,
        ],
        compiler_params=pltpu.CompilerParams(
            collective_id=0,
            vmem_limit_bytes=62 * 1024 * 1024,
        ),
    )(x, w_mat)
    return out


# device time: 629684 ns/iter; 2.2341x vs baseline; 1.1457x over previous
import functools

import jax
import jax.numpy as jnp
from jax import lax
from jax.experimental import pallas as pl
from jax.experimental.pallas import tpu as pltpu

N_DEV = 4
N_STEP = N_DEV - 1
K_TILES = 2
N_SUB = 2


def _body(x_ref, w_ref, out_ref, recv_hbm, send_hbm,
          comp_r, comp_l, stage, xt, wt,
          send_sems, recv_sems, local_sem):
    my = lax.axis_index("i")
    left = (my - 1) % N_DEV
    right = (my + 1) % N_DEV
    m_sub, n_half = comp_r.shape
    m_per = m_sub * N_SUB
    k_loc = x_ref.shape[1]
    k_tile = k_loc // K_TILES

    barrier_sem = pltpu.get_barrier_semaphore()
    for nbr in (left, right):
        pl.semaphore_signal(barrier_sem, inc=1, device_id=(nbr,),
                            device_id_type=pl.DeviceIdType.MESH)
    pl.semaphore_wait(barrier_sem, 2)

    def copy(src, dst):
        cp = pltpu.make_async_copy(src, dst, local_sem)
        cp.start()
        cp.wait()

    def rdma(d, st, nbr, sub):
        return pltpu.make_async_remote_copy(
            src_ref=send_hbm.at[d, sub],
            dst_ref=recv_hbm.at[d, st],
            send_sem=send_sems.at[d, st],
            recv_sem=recv_sems.at[d, st],
            device_id=(nbr,),
            device_id_type=pl.DeviceIdType.MESH,
        )

    def direction(d, t, sub, nbr, c, col_off, comp):
        row0 = c * m_per + sub * m_sub
        for kt in range(K_TILES):
            copy(x_ref.at[pl.ds(row0, m_sub), pl.ds(kt * k_tile, k_tile)], xt)
            copy(w_ref.at[pl.ds(kt * k_tile, k_tile), pl.ds(col_off, n_half)],
                 wt)
            prod = jnp.dot(xt[...], wt[...],
                           preferred_element_type=jnp.float32)
            if kt == 0:
                comp[...] = prod
            else:
                comp[...] = comp[...] + prod

        st_t = t * N_SUB + sub

        @pl.when(t >= 0)
        def _():
            rdma(d, st_t, nbr, sub).wait()
            copy(recv_hbm.at[d, st_t], stage)

        @pl.when((t >= 0) & (t < N_STEP - 1))
        def _():
            comp[...] = comp[...] + stage[...]

        @pl.when(t == N_STEP - 1)
        def _():
            comp[...] = jnp.maximum(comp[...] + stage[...], 0.0)
            copy(comp, out_ref.at[pl.ds(sub * m_sub, m_sub),
                                  pl.ds(col_off, n_half)])

        @pl.when(t < N_STEP - 1)
        def _():
            copy(comp, send_hbm.at[d, sub])
            rdma(d, st_t + N_SUB, nbr, sub).start()

    def iteration(i, carry):
        s = i // N_SUB
        sub = i - s * N_SUB
        t = s - 1
        direction(0, t, sub, right, (my - 2 - t) % N_DEV, 0, comp_r)
        direction(1, t, sub, left, (my + 2 + t) % N_DEV, n_half, comp_l)
        return carry

    lax.fori_loop(0, (N_STEP + 1) * N_SUB, iteration, 0)

    @functools.partial(pl.run_scoped, second=pltpu.SemaphoreType.REGULAR)
    def _(second):
        for nbr in (left, right):
            pl.semaphore_signal(second, inc=1, device_id=(nbr,),
                                device_id_type=pl.DeviceIdType.MESH)
        pl.semaphore_wait(second, 2)


def kernel(x, w_mat):
    m, k_loc = x.shape
    _, n_out = w_mat.shape
    m_per = m // N_DEV
    m_sub = m_per // N_SUB
    n_half = n_out // 2
    k_tile = k_loc // K_TILES
    n_st = N_STEP * N_SUB

    out, _, _ = pl.pallas_call(
        _body,
        out_shape=[
            jax.ShapeDtypeStruct((m_per, n_out), jnp.float32),
            jax.ShapeDtypeStruct((2, n_st, m_sub, n_half), jnp.float32),
            jax.ShapeDtypeStruct((2, N_SUB, m_sub, n_half), jnp.float32),
        ],
        in_specs=[
            pl.BlockSpec(memory_space=pl.ANY),
            pl.BlockSpec(memory_space=pl.ANY),
        ],
        out_specs=[
            pl.BlockSpec(memory_space=pl.ANY),
            pl.BlockSpec(memory_space=pl.ANY),
            pl.BlockSpec(memory_space=pl.ANY),
        ],
        scratch_shapes=[
            pltpu.VMEM((m_sub, n_half), jnp.float32),
            pltpu.VMEM((m_sub, n_half), jnp.float32),
            pltpu.VMEM((m_sub, n_half), jnp.float32),
            pltpu.VMEM((m_sub, k_tile), jnp.float32),
            pltpu.VMEM((k_tile, n_half), jnp.float32),
            pltpu.SemaphoreType.DMA((2, n_st)),
            pltpu.SemaphoreType.DMA((2, n_st)),
            pltpu.SemaphoreType.DMA,
        ],
        compiler_params=pltpu.CompilerParams(
            collective_id=0,
            vmem_limit_bytes=62 * 1024 * 1024,
        ),
    )(x, w_mat)
    return out


# device time: 607740 ns/iter; 2.3147x vs baseline; 1.0361x over previous
import functools

import jax
import jax.numpy as jnp
from jax import lax
from jax.experimental import pallas as pl
from jax.experimental.pallas import tpu as pltpu

N_DEV = 4
N_STEP = N_DEV - 1
K_TILES = 2
N_SUB = 4


def _body(x_ref, w_ref, out_ref, recv_hbm, send_hbm,
          comp_r, comp_l, stage, xt, wc,
          send_sems, recv_sems, local_sem):
    my = lax.axis_index("i")
    left = (my - 1) % N_DEV
    right = (my + 1) % N_DEV
    m_sub, n_half = comp_r.shape
    m_per = m_sub * N_SUB
    k_loc = x_ref.shape[1]
    k_tile = k_loc // K_TILES

    barrier_sem = pltpu.get_barrier_semaphore()
    for nbr in (left, right):
        pl.semaphore_signal(barrier_sem, inc=1, device_id=(nbr,),
                            device_id_type=pl.DeviceIdType.MESH)
    pl.semaphore_wait(barrier_sem, 2)

    def copy(src, dst):
        cp = pltpu.make_async_copy(src, dst, local_sem)
        cp.start()
        cp.wait()

    def load_w_cache(d, col_off):
        for kt in range(K_TILES):
            copy(w_ref.at[pl.ds(kt * k_tile, k_tile), pl.ds(col_off, n_half)],
                 wc.at[d, kt])

    def rdma(d, st, nbr, sub):
        return pltpu.make_async_remote_copy(
            src_ref=send_hbm.at[d, sub],
            dst_ref=recv_hbm.at[d, st],
            send_sem=send_sems.at[d, st],
            recv_sem=recv_sems.at[d, st],
            device_id=(nbr,),
            device_id_type=pl.DeviceIdType.MESH,
        )

    def direction(d, t, sub, nbr, c, col_off, comp):
        row0 = c * m_per + sub * m_sub
        for kt in range(K_TILES):
            copy(x_ref.at[pl.ds(row0, m_sub), pl.ds(kt * k_tile, k_tile)], xt)
            prod = jnp.dot(xt[...], wc.at[d, kt][...],
                           preferred_element_type=jnp.float32)
            if kt == 0:
                comp[...] = prod
            else:
                comp[...] = comp[...] + prod

        st_t = t * N_SUB + sub

        @pl.when(t >= 0)
        def _():
            rdma(d, st_t, nbr, sub).wait()
            copy(recv_hbm.at[d, st_t], stage)

        @pl.when((t >= 0) & (t < N_STEP - 1))
        def _():
            comp[...] = comp[...] + stage[...]

        @pl.when(t == N_STEP - 1)
        def _():
            comp[...] = jnp.maximum(comp[...] + stage[...], 0.0)
            copy(comp, out_ref.at[pl.ds(sub * m_sub, m_sub),
                                  pl.ds(col_off, n_half)])

        @pl.when(t < N_STEP - 1)
        def _():
            copy(comp, send_hbm.at[d, sub])
            rdma(d, st_t + N_SUB, nbr, sub).start()

    load_w_cache(0, 0)

    n_half_off = n_half

    def iteration(i, carry):
        s = i // N_SUB
        sub = i - s * N_SUB
        t = s - 1
        direction(0, t, sub, right, (my - 2 - t) % N_DEV, 0, comp_r)

        @pl.when(i == 0)
        def _():
            load_w_cache(1, n_half_off)

        direction(1, t, sub, left, (my + 2 + t) % N_DEV, n_half_off, comp_l)
        return carry

    lax.fori_loop(0, (N_STEP + 1) * N_SUB, iteration, 0)

    @functools.partial(pl.run_scoped, second=pltpu.SemaphoreType.REGULAR)
    def _(second):
        for nbr in (left, right):
            pl.semaphore_signal(second, inc=1, device_id=(nbr,),
                                device_id_type=pl.DeviceIdType.MESH)
        pl.semaphore_wait(second, 2)


def kernel(x, w_mat):
    m, k_loc = x.shape
    _, n_out = w_mat.shape
    m_per = m // N_DEV
    m_sub = m_per // N_SUB
    n_half = n_out // 2
    k_tile = k_loc // K_TILES
    n_st = N_STEP * N_SUB

    out, _, _ = pl.pallas_call(
        _body,
        out_shape=[
            jax.ShapeDtypeStruct((m_per, n_out), jnp.float32),
            jax.ShapeDtypeStruct((2, n_st, m_sub, n_half), jnp.float32),
            jax.ShapeDtypeStruct((2, N_SUB, m_sub, n_half), jnp.float32),
        ],
        in_specs=[
            pl.BlockSpec(memory_space=pl.ANY),
            pl.BlockSpec(memory_space=pl.ANY),
        ],
        out_specs=[
            pl.BlockSpec(memory_space=pl.ANY),
            pl.BlockSpec(memory_space=pl.ANY),
            pl.BlockSpec(memory_space=pl.ANY),
        ],
        scratch_shapes=[
            pltpu.VMEM((m_sub, n_half), jnp.float32),
            pltpu.VMEM((m_sub, n_half), jnp.float32),
            pltpu.VMEM((m_sub, n_half), jnp.float32),
            pltpu.VMEM((m_sub, k_tile), jnp.float32),
            pltpu.VMEM((2, K_TILES, k_tile, n_half), jnp.float32),
            pltpu.SemaphoreType.DMA((2, n_st)),
            pltpu.SemaphoreType.DMA((2, n_st)),
            pltpu.SemaphoreType.DMA,
        ],
        compiler_params=pltpu.CompilerParams(
            collective_id=0,
            vmem_limit_bytes=62 * 1024 * 1024,
        ),
    )(x, w_mat)
    return out


# device time: 336026 ns/iter; 4.1864x vs baseline; 1.8086x over previous
import functools

import jax
import jax.numpy as jnp
from jax import lax
from jax.experimental import pallas as pl
from jax.experimental.pallas import tpu as pltpu

N_DEV = 4
N_STEP = N_DEV - 1
K_TILES = 1
N_SUB = 4


def _body(x_ref, w_ref, out_ref, recv_hbm, send_hbm,
          comp_r, comp_l, stage, sendb, xt0, xt1, wc,
          send_sems, recv_sems, local_sem, xsems):
    my = lax.axis_index("i")
    left = (my - 1) % N_DEV
    right = (my + 1) % N_DEV
    m_sub, n_half = comp_r.shape
    m_per = m_sub * N_SUB
    k_loc = x_ref.shape[1]
    k_tile = k_loc // K_TILES

    barrier_sem = pltpu.get_barrier_semaphore()
    for nbr in (left, right):
        pl.semaphore_signal(barrier_sem, inc=1, device_id=(nbr,),
                            device_id_type=pl.DeviceIdType.MESH)
    pl.semaphore_wait(barrier_sem, 2)

    def copy(src, dst):
        cp = pltpu.make_async_copy(src, dst, local_sem)
        cp.start()
        cp.wait()

    def load_w_cache(d, col_off):
        for kt in range(K_TILES):
            copy(w_ref.at[pl.ds(kt * k_tile, k_tile), pl.ds(col_off, n_half)],
                 wc.at[d, kt])

    def rdma(d, st, nbr, sub):
        return pltpu.make_async_remote_copy(
            src_ref=send_hbm.at[d, sub],
            dst_ref=recv_hbm.at[d, st],
            send_sem=send_sems.at[d, st],
            recv_sem=recv_sems.at[d, st],
            device_id=(nbr,),
            device_id_type=pl.DeviceIdType.MESH,
        )

    def xload(c, sub, xt, xsem):
        row0 = c * m_per + sub * m_sub
        cp = pltpu.make_async_copy(
            x_ref.at[pl.ds(row0, m_sub), :], xt, xsem)
        cp.start()
        return cp

    def direction(d, t, sub, nbr, col_off, comp, xt, xcp):
        xcp.wait()
        acc = jnp.dot(xt[...], wc.at[d, 0][...],
                      preferred_element_type=jnp.float32)

        st_t = t * N_SUB + sub

        @pl.when(t >= 0)
        def _():
            rdma(d, st_t, nbr, sub).wait()
            copy(recv_hbm.at[d, st_t], stage)

        @pl.when(t == -1)
        def _():
            sendb[...] = acc.astype(jnp.bfloat16)

        @pl.when((t >= 0) & (t < N_STEP - 1))
        def _():
            sendb[...] = (acc + stage[...].astype(jnp.float32)
                          ).astype(jnp.bfloat16)

        @pl.when(t < N_STEP - 1)
        def _():
            copy(sendb, send_hbm.at[d, sub])
            rdma(d, st_t + N_SUB, nbr, sub).start()

        @pl.when(t == N_STEP - 1)
        def _():
            comp[...] = jnp.maximum(
                acc + stage[...].astype(jnp.float32), 0.0)
            copy(comp, out_ref.at[pl.ds(sub * m_sub, m_sub),
                                  pl.ds(col_off, n_half)])

    load_w_cache(0, 0)

    n_half_off = n_half

    def iteration(i, carry):
        s = i // N_SUB
        sub = i - s * N_SUB
        t = s - 1
        xcp_r = xload((my - 2 - t) % N_DEV, sub, xt0, xsems.at[0])
        xcp_l = xload((my + 2 + t) % N_DEV, sub, xt1, xsems.at[1])
        direction(0, t, sub, right, 0, comp_r, xt0, xcp_r)

        @pl.when(i == 0)
        def _():
            load_w_cache(1, n_half_off)

        direction(1, t, sub, left, n_half_off, comp_l, xt1, xcp_l)
        return carry

    lax.fori_loop(0, (N_STEP + 1) * N_SUB, iteration, 0)

    @functools.partial(pl.run_scoped, second=pltpu.SemaphoreType.REGULAR)
    def _(second):
        for nbr in (left, right):
            pl.semaphore_signal(second, inc=1, device_id=(nbr,),
                                device_id_type=pl.DeviceIdType.MESH)
        pl.semaphore_wait(second, 2)


def kernel(x, w_mat):
    m, k_loc = x.shape
    _, n_out = w_mat.shape
    m_per = m // N_DEV
    m_sub = m_per // N_SUB
    n_half = n_out // 2
    k_tile = k_loc // K_TILES
    n_st = N_STEP * N_SUB

    out, _, _ = pl.pallas_call(
        _body,
        out_shape=[
            jax.ShapeDtypeStruct((m_per, n_out), jnp.float32),
            jax.ShapeDtypeStruct((2, n_st, m_sub, n_half), jnp.bfloat16),
            jax.ShapeDtypeStruct((2, N_SUB, m_sub, n_half), jnp.bfloat16),
        ],
        in_specs=[
            pl.BlockSpec(memory_space=pl.ANY),
            pl.BlockSpec(memory_space=pl.ANY),
        ],
        out_specs=[
            pl.BlockSpec(memory_space=pl.ANY),
            pl.BlockSpec(memory_space=pl.ANY),
            pl.BlockSpec(memory_space=pl.ANY),
        ],
        scratch_shapes=[
            pltpu.VMEM((m_sub, n_half), jnp.float32),
            pltpu.VMEM((m_sub, n_half), jnp.float32),
            pltpu.VMEM((m_sub, n_half), jnp.bfloat16),
            pltpu.VMEM((m_sub, n_half), jnp.bfloat16),
            pltpu.VMEM((m_sub, k_tile), jnp.float32),
            pltpu.VMEM((m_sub, k_tile), jnp.float32),
            pltpu.VMEM((2, K_TILES, k_tile, n_half), jnp.float32),
            pltpu.SemaphoreType.DMA((2, n_st)),
            pltpu.SemaphoreType.DMA((2, n_st)),
            pltpu.SemaphoreType.DMA,
            pltpu.SemaphoreType.DMA((K_TILES,)),
        ],
        compiler_params=pltpu.CompilerParams(
            collective_id=0,
            vmem_limit_bytes=62 * 1024 * 1024,
        ),
    )(x, w_mat)
    return out
